# baseline (speedup 1.0000x reference)
import jax
import jax.numpy as jnp
from jax import lax
from jax.experimental import pallas as pl
from jax.experimental.pallas import tpu as pltpu
from jax.experimental.pallas import tpu_sc as plsc

_EMB = 64
_D2 = 2 * _EMB
_B = 16384
_NC, _NS, _L = 2, 16, 16
_NW = _NC * _NS
_BPW = _B // _NW
_CH = 128
_NCHUNK = _BPW // _CH


def _sc_body(head_hbm, rel_hbm, tail_hbm, ent_hbm, relemb_hbm, out_hbm,
             ht_idx, r_idx, ht_rows, r_rows, out_v, sems):
    wid = lax.axis_index("s") * _NC + lax.axis_index("c")
    base = wid * _BPW
    iota = lax.iota(jnp.int32, 16)
    lane15 = iota == 15
    for ch in range(_NCHUNK):
        off = base + ch * _CH
        pltpu.sync_copy(head_hbm.at[pl.ds(off, _CH)],
                        ht_idx.at[pl.ds(ch * 2 * _CH, _CH)])
        pltpu.sync_copy(tail_hbm.at[pl.ds(off, _CH)],
                        ht_idx.at[pl.ds(ch * 2 * _CH + _CH, _CH)])
    pltpu.sync_copy(rel_hbm.at[pl.ds(base, _BPW)], r_idx)

    def start(ch):
        slot = ch & 1
        return (
            pltpu.async_copy(ent_hbm.at[ht_idx.at[pl.ds(ch * 2 * _CH, 2 * _CH)]],
                             ht_rows.at[slot], sems.at[slot]),
            pltpu.async_copy(relemb_hbm.at[r_idx.at[pl.ds(ch * _CH, _CH)]],
                             r_rows.at[slot], sems.at[slot]),
        )

    pending = start(0)
    for ch in range(_NCHUNK):
        slot = ch & 1
        for c in pending:
            c.wait()
        if ch + 1 < _NCHUNK:
            pending = start(ch + 1)

        @plsc.parallel_loop(0, _CH, unroll=4)
        def _rows(i):
            partial = jnp.zeros((16,), jnp.float32)
            for c in range(_EMB // 16):
                re = pl.ds(c * 16, 16)
                im = pl.ds(_EMB + c * 16, 16)
                hr = ht_rows[slot, i, re]
                hi = ht_rows[slot, i, im]
                rr = r_rows[slot, i, re]
                ri = r_rows[slot, i, im]
                tr = ht_rows[slot, _CH + i, re]
                ti = ht_rows[slot, _CH + i, im]
                a = hr * rr + hi * ri
                b = hr * ri - hi * rr
                partial = partial + (a * tr - b * ti)
            total = plsc.cumsum(partial)
            pos = jnp.full((16,), ch * _CH + i, jnp.int32)
            plsc.store_scatter(out_v, [pos], total, mask=lane15)

    pltpu.sync_copy(out_v, out_hbm.at[pl.ds(base, _BPW)])


@jax.jit
def kernel(head_ids, relation_ids, tail_ids, entity_emb, relation_emb):
    k = pl.kernel(
        _sc_body,
        out_type=jax.ShapeDtypeStruct((_B,), jnp.float32),
        mesh=plsc.VectorSubcoreMesh(core_axis_name="c", subcore_axis_name="s"),
        compiler_params=pltpu.CompilerParams(needs_layout_passes=False),
        scratch_types=[
            pltpu.VMEM((2 * _BPW,), jnp.int32),
            pltpu.VMEM((_BPW,), jnp.int32),
            pltpu.VMEM((2, 2 * _CH, _D2), jnp.float32),
            pltpu.VMEM((2, _CH, _D2), jnp.float32),
            pltpu.VMEM((_BPW,), jnp.float32),
            pltpu.SemaphoreType.DMA((2,)),
        ],
    )
    return k(head_ids, relation_ids, tail_ids, entity_emb, relation_emb)
